# baseline (speedup 1.0000x reference)
import functools

import jax
import jax.numpy as jnp
from jax import lax
from jax.experimental import pallas as pl
from jax.experimental.pallas import tpu as pltpu
from jax.experimental.pallas import tpu_sc as plsc

N = 10000
E = 320000
D = 128
H = 2 * D
L = 3

NC = 2
NS = 16
EP = E // (NC * NS)
EK = 125
NCH = EP // EK
NB = 3
NCH_MAIN = (NCH // NB) * NB
RPT = 624
RPT_LAST = N - RPT * (NS - 1)
R_LAST0 = RPT * (NS - 1)


def _sc_aggregate(h, src, dst):
    mesh = plsc.VectorSubcoreMesh(core_axis_name="c", subcore_axis_name="s")

    @functools.partial(
        pl.kernel,
        out_type=jax.ShapeDtypeStruct((NC, N, D), jnp.float32),
        mesh=mesh,
        scratch_types=[
            [pltpu.VMEM((EK,), jnp.int32) for _ in range(NB)],
            [pltpu.VMEM((EK,), jnp.int32) for _ in range(NB)],
            [pltpu.VMEM((EK, D), jnp.float32) for _ in range(NB)],
            pltpu.VMEM_SHARED((N, D), jnp.float32),
            [pltpu.SemaphoreType.DMA for _ in range(NB)],
            [pltpu.SemaphoreType.DMA for _ in range(NB)],
            [pltpu.SemaphoreType.DMA for _ in range(NB)],
        ],
    )
    def sc_kernel(h_hbm, src_hbm, dst_hbm, out_hbm, srcb, dstb,
                  rows, acc, semsi, semd, sem):
        c = lax.axis_index("c")
        s = lax.axis_index("s")
        r0 = s * RPT
        tile = c * NS + s

        src_t = src_hbm.at[tile]
        dst_t = dst_hbm.at[tile]

        @pl.when(s < NS - 1)
        def _init_main():
            pltpu.sync_copy(h_hbm.at[pl.ds(r0, RPT)], acc.at[pl.ds(r0, RPT)])

        @pl.when(s == NS - 1)
        def _init_last():
            pltpu.sync_copy(h_hbm.at[pl.ds(R_LAST0, RPT_LAST)],
                            acc.at[pl.ds(R_LAST0, RPT_LAST)])

        plsc.subcore_barrier()

        def issue_idx(k, b):
            pltpu.async_copy(src_t.at[k], srcb[b], semsi[b])
            pltpu.async_copy(dst_t.at[k], dstb[b], semd[b])

        def issue_gather(k, b):
            pltpu.make_async_copy(src_t.at[k], srcb[b], semsi[b]).wait()
            pltpu.async_copy(h_hbm.at[srcb[b]], rows[b], sem[b])

        for b in range(NB):
            issue_idx(b, b)
        for b in range(NB):
            issue_gather(b, b)

        @pl.loop(0, NCH_MAIN, step=NB)
        def _edge_chunk(j):
            for b in range(NB):
                k = j + b
                pltpu.make_async_copy(h_hbm.at[srcb[b]], rows[b], sem[b]).wait()

                @pl.when(k + NB < NCH)
                def _prefetch_src_idx():
                    pltpu.async_copy(src_t.at[k + NB], srcb[b], semsi[b])

                pltpu.make_async_copy(dst_t.at[k], dstb[b], semd[b]).wait()
                pltpu.sync_copy(rows[b], acc.at[dstb[b]], add=True)

                @pl.when(k + NB < NCH)
                def _refill_gather():
                    pltpu.async_copy(dst_t.at[k + NB], dstb[b], semd[b])
                    issue_gather(k + NB, b)

        for b in range(NCH - NCH_MAIN):
            pltpu.make_async_copy(h_hbm.at[srcb[b]], rows[b], sem[b]).wait()
            pltpu.make_async_copy(dst_t.at[NCH_MAIN + b], dstb[b], semd[b]).wait()
            pltpu.sync_copy(rows[b], acc.at[dstb[b]], add=True)

        plsc.subcore_barrier()

        @pl.when(s < NS - 1)
        def _out_main():
            pltpu.sync_copy(acc.at[pl.ds(r0, RPT)],
                            out_hbm.at[c].at[pl.ds(r0, RPT)])

        @pl.when(s == NS - 1)
        def _out_last():
            pltpu.sync_copy(acc.at[pl.ds(R_LAST0, RPT_LAST)],
                            out_hbm.at[c].at[pl.ds(R_LAST0, RPT_LAST)])

    return sc_kernel(h, src, dst)


def _tc_embed(x, W0, b0):
    def body(x_ref, w_ref, b_ref, o_ref):
        o_ref[...] = (
            jnp.dot(x_ref[...], w_ref[...], preferred_element_type=jnp.float32)
            + b_ref[...]
        )

    return pl.pallas_call(
        body,
        out_shape=jax.ShapeDtypeStruct((N, D), jnp.float32),
    )(x, W0, b0.reshape(1, D))


def _tc_layer(h, parts, W1, b1, g1, be1, W2, b2, g, be):
    def body(h_ref, p_ref, w1_ref, b1_ref, g1_ref, be1_ref, w2_ref, b2_ref,
             g_ref, be_ref, o_ref):
        hv = h_ref[...]
        z0 = p_ref[0] + p_ref[1] - hv
        z = jnp.dot(z0, w1_ref[...], preferred_element_type=jnp.float32) + b1_ref[...]
        m = jnp.mean(z, axis=0, keepdims=True)
        v = jnp.mean((z - m) * (z - m), axis=0, keepdims=True)
        z = g1_ref[...] * (z - m) * lax.rsqrt(v + 1e-5) + be1_ref[...]
        z = jnp.maximum(z, 0.0)
        z = jnp.dot(z, w2_ref[...], preferred_element_type=jnp.float32) + b2_ref[...]
        hn = z + hv
        m2 = jnp.mean(hn, axis=0, keepdims=True)
        v2 = jnp.mean((hn - m2) * (hn - m2), axis=0, keepdims=True)
        o_ref[...] = g_ref[...] * (hn - m2) * lax.rsqrt(v2 + 1e-5) + be_ref[...]

    return pl.pallas_call(
        body,
        out_shape=jax.ShapeDtypeStruct((N, D), jnp.float32),
    )(h, parts, W1, b1.reshape(1, H), g1.reshape(1, H), be1.reshape(1, H),
      W2, b2.reshape(1, D), g.reshape(1, D), be.reshape(1, D))


def kernel(pre_node_emb, edge_index, W0, b0, W1s, b1s, g1s, be1s, W2s, b2s, gs, bes):
    x = pre_node_emb[0]
    src = edge_index[0, :, 0].reshape(NC * NS, NCH, EK)
    dst = edge_index[0, :, 1].reshape(NC * NS, NCH, EK)
    h = _tc_embed(x, W0, b0)
    for i in range(L):
        parts = _sc_aggregate(h, src, dst)
        h = _tc_layer(h, parts, W1s[i], b1s[i], g1s[i], be1s[i],
                      W2s[i], b2s[i], gs[i], bes[i])
    return h
